# Initial kernel scaffold; baseline (speedup 1.0000x reference)
import jax
import jax.numpy as jnp
from jax.experimental import pallas as pl


def kernel(h_src, h_dst, edge_index, seed_score):
    raise NotImplementedError("write your pallas kernel here")


# speedup vs baseline: 1.1495x; 1.1495x over previous
import functools

import jax
import jax.numpy as jnp
from jax import lax
from jax.experimental import pallas as pl
from jax.experimental.pallas import tpu as pltpu
from jax.experimental.pallas import tpu_sc as plsc

N_NODES = 10000
N_EDGES = 320000
D_FEAT = 128

NUM_CORES = 2
NUM_SUBCORES = 16
NUM_WORKERS = NUM_CORES * NUM_SUBCORES
E_PER_W = N_EDGES // NUM_WORKERS
BLK = 80
NBLK = E_PER_W // BLK
GRP = BLK // 16
FU = 8


def _sc_call(h_src, h_dst, src_i, dst_i, seed):
    mesh = plsc.VectorSubcoreMesh(core_axis_name="c", subcore_axis_name="s")

    @functools.partial(
        pl.kernel,
        out_type=jax.ShapeDtypeStruct((N_EDGES,), jnp.float32),
        mesh=mesh,
        compiler_params=pltpu.CompilerParams(needs_layout_passes=False),
        scratch_types=[
            pltpu.VMEM((E_PER_W,), jnp.int32),
            pltpu.VMEM((E_PER_W,), jnp.int32),
            pltpu.VMEM((E_PER_W,), jnp.float32),
            pltpu.VMEM((E_PER_W,), jnp.float32),
            pltpu.VMEM((BLK, D_FEAT), jnp.float32),
            pltpu.VMEM((BLK, D_FEAT), jnp.float32),
            pltpu.SemaphoreType.DMA,
        ],
    )
    def k(hs, hd, si, di, sc, out, si_v, di_v, sc_v, out_v, ru, rv, sem):
        wid = lax.axis_index("s") * NUM_CORES + lax.axis_index("c")
        base = wid * E_PER_W
        pltpu.sync_copy(si.at[pl.ds(base, E_PER_W)], si_v)
        pltpu.sync_copy(di.at[pl.ds(base, E_PER_W)], di_v)
        pltpu.sync_copy(sc.at[pl.ds(base, E_PER_W)], sc_v)

        lane = lax.iota(jnp.int32, 16)
        zeros = jnp.zeros((16,), jnp.float32)

        @pl.loop(0, NBLK)
        def _blk_loop(blk):
            off = blk * BLK
            cu = pltpu.async_copy(hs.at[si_v.at[pl.ds(off, BLK)]], ru, sem)
            cv = pltpu.async_copy(hd.at[di_v.at[pl.ds(off, BLK)]], rv, sem)
            cu.wait()
            cv.wait()

            row_vecs = [lane + 16 * g for g in range(GRP)]

            def body(fc, accs):
                f0 = fc * FU
                accs = list(accs)
                for j in range(FU):
                    col = jnp.full((16,), 0, jnp.int32) + (f0 + j)
                    for g in range(GRP):
                        u = plsc.load_gather(ru, [row_vecs[g], col])
                        v = plsc.load_gather(rv, [row_vecs[g], col])
                        accs[g] = accs[g] + u * v
                return tuple(accs)

            accs = lax.fori_loop(0, D_FEAT // FU, body,
                                 tuple(zeros for _ in range(GRP)))
            for g in range(GRP):
                pos = off + g * 16
                out_v[pl.ds(pos, 16)] = accs[g] + sc_v[pl.ds(pos, 16)]

        pltpu.sync_copy(out_v, out.at[pl.ds(base, E_PER_W)])

    return k(h_src, h_dst, src_i, dst_i, seed)


@jax.jit
def kernel(h_src, h_dst, edge_index, seed_score):
    src_i = edge_index[0].astype(jnp.int32)
    dst_i = edge_index[1].astype(jnp.int32)
    return _sc_call(h_src, h_dst, src_i, dst_i, seed_score)
